# baseline (speedup 1.0000x reference)
import functools

import jax
import jax.numpy as jnp
from jax import lax
from jax.experimental import pallas as pl
from jax.experimental.pallas import tpu as pltpu
from jax.experimental.pallas import tpu_sc as plsc

_NUM_CLASSES = 1000000
_ROWS = _NUM_CLASSES + 1
_OUT_DIM = 64
_BATCH = 16384
_L = 16
_NC = 2
_NS = 16
_NW = _NC * _NS
_B_PER_W = _BATCH // _NW
_NG = _B_PER_W // _L
_NCHUNK = 6
_CROWS = 166664

_mesh = plsc.VectorSubcoreMesh(core_axis_name="c", subcore_axis_name="s")


def _make_embed(cbase, csize, has_null):
    @functools.partial(
        pl.kernel,
        mesh=_mesh,
        out_type=jax.ShapeDtypeStruct((_BATCH, _OUT_DIM), jnp.float32),
        scratch_types=[
            pltpu.VMEM((_B_PER_W,), jnp.int32),
            pltpu.VMEM((_B_PER_W,), jnp.int32),
            pltpu.VMEM((_B_PER_W,), jnp.int32),
            pltpu.VMEM((_B_PER_W,), jnp.int32),
            pltpu.VMEM((_L,), jnp.int32),
            pltpu.VMEM((1, _OUT_DIM), jnp.float32),
            pltpu.VMEM((_B_PER_W, _OUT_DIM), jnp.float32),
            pltpu.SemaphoreType.DMA,
            pltpu.SemaphoreType.DMA,
        ],
        compiler_params=pltpu.CompilerParams(needs_layout_passes=False),
    )
    def _embed(labels_hbm, train_hbm, drop_hbm, chunk_hbm, null_hbm, out_hbm,
               idx_v, drop_v, val_v, dr_v, train_v, null_v, rows_v,
               sem, sem2):
        wid = lax.axis_index("s") * _NC + lax.axis_index("c")
        base = wid * _B_PER_W
        pltpu.sync_copy(labels_hbm.at[pl.ds(base, _B_PER_W)], idx_v)
        pltpu.sync_copy(drop_hbm.at[pl.ds(base, _B_PER_W)], drop_v)
        pltpu.sync_copy(train_hbm, train_v)
        if has_null:
            pltpu.sync_copy(null_hbm, null_v)
        trn = train_v[...]
        one16 = jnp.full((_L,), 1, jnp.int32)
        zero16 = jnp.full((_L,), 0, jnp.int32)
        zerof16 = jnp.zeros((_L,), jnp.float32)
        cnt = jnp.int32(0)
        for i in range(_NG):
            sl = pl.ds(i * _L, _L)
            dropped = (trn != 0) & (drop_v[sl] != 0)
            local = idx_v[sl] - cbase
            valid = (~dropped) & (local >= 0) & (local < csize)
            idx_v[sl] = jnp.where(valid, local, 0)
            val_v[sl] = jnp.where(valid, one16, zero16)
            dr_v[sl] = jnp.where(dropped, one16, zero16)
            cnt = cnt + plsc.all_reduce_population_count(valid)[0]

        @pl.loop(0, _B_PER_W)
        def _(j):
            for m in range(_OUT_DIM // _L):
                rows_v[j, pl.ds(m * _L, _L)] = zerof16

        @pl.loop(0, _NG)
        def _(g):
            local = idx_v[pl.ds(g * _L, _L)]
            valid = val_v[pl.ds(g * _L, _L)]
            for k in range(_L):
                @pl.when(valid[k] != 0)
                def _():
                    pltpu.async_copy(
                        chunk_hbm.at[pl.ds(local[k], 1), :],
                        rows_v.at[pl.ds(g * _L + k, 1), :], sem)

        if has_null:
            nulls = [null_v[0, pl.ds(m * _L, _L)]
                     for m in range(_OUT_DIM // _L)]

            @pl.loop(0, _NG)
            def _(g):
                dr = dr_v[pl.ds(g * _L, _L)]
                for k in range(_L):
                    @pl.when(dr[k] != 0)
                    def _():
                        for m in range(_OUT_DIM // _L):
                            rows_v[g * _L + k, pl.ds(m * _L, _L)] = nulls[m]

        @pl.loop(0, _B_PER_W)
        def _(j):
            @pl.when(j < cnt)
            def _():
                pltpu.make_async_copy(
                    chunk_hbm.at[pl.ds(0, 1), :],
                    rows_v.at[pl.ds(0, 1), :], sem).wait()

        pltpu.async_copy(rows_v, out_hbm.at[pl.ds(base, _B_PER_W)],
                         sem2).wait()

    return _embed


def kernel(labels, train, force_drop_ids, table):
    labels32 = labels.astype(jnp.int32)
    drop32 = force_drop_ids.astype(jnp.int32)
    train_vec = jnp.full((_L,), jnp.asarray(train, dtype=jnp.int32))
    nullrow = jnp.take(table, jnp.full((1,), _NUM_CLASSES, jnp.int32),
                       axis=0)
    sizes = [_CROWS] * (_NCHUNK - 1) + [_NUM_CLASSES - (_NCHUNK - 1) * _CROWS]
    out = None
    cbase = 0
    for c in range(_NCHUNK):
        csize = sizes[c]
        has_null = c == _NCHUNK - 1
        chunk = lax.slice(table, (cbase, 0), (cbase + csize, _OUT_DIM))
        part = _make_embed(cbase, csize, has_null)(
            labels32, train_vec, drop32, chunk, nullrow)
        out = part if out is None else out + part
        cbase += csize
    return out


# speedup vs baseline: 1.0547x; 1.0547x over previous
import functools

import jax
import jax.numpy as jnp
from jax import lax
from jax.experimental import pallas as pl
from jax.experimental.pallas import tpu as pltpu
from jax.experimental.pallas import tpu_sc as plsc

_NUM_CLASSES = 1000000
_ROWS = _NUM_CLASSES + 1
_OUT_DIM = 64
_BATCH = 16384
_L = 16
_NC = 2
_NS = 16
_NW = _NC * _NS
_B_PER_W = _BATCH // _NW
_NG = _B_PER_W // _L
_NCHUNK = 3
_CROWS = 333336

_mesh = plsc.VectorSubcoreMesh(core_axis_name="c", subcore_axis_name="s")


def _make_embed(cbase, csize, has_null):
    @functools.partial(
        pl.kernel,
        mesh=_mesh,
        out_type=jax.ShapeDtypeStruct((_BATCH, _OUT_DIM), jnp.float32),
        scratch_types=[
            pltpu.VMEM((_B_PER_W,), jnp.int32),
            pltpu.VMEM((_B_PER_W,), jnp.int32),
            pltpu.VMEM((_B_PER_W,), jnp.int32),
            pltpu.VMEM((_B_PER_W,), jnp.int32),
            pltpu.VMEM((_L,), jnp.int32),
            pltpu.VMEM((1, _OUT_DIM), jnp.float32),
            pltpu.VMEM((_B_PER_W, _OUT_DIM), jnp.float32),
            pltpu.SemaphoreType.DMA,
            pltpu.SemaphoreType.DMA,
        ],
        compiler_params=pltpu.CompilerParams(needs_layout_passes=False),
    )
    def _embed(labels_hbm, train_hbm, drop_hbm, chunk_hbm, null_hbm, out_hbm,
               idx_v, drop_v, val_v, dr_v, train_v, null_v, rows_v,
               sem, sem2):
        wid = lax.axis_index("s") * _NC + lax.axis_index("c")
        base = wid * _B_PER_W
        pltpu.sync_copy(labels_hbm.at[pl.ds(base, _B_PER_W)], idx_v)
        pltpu.sync_copy(drop_hbm.at[pl.ds(base, _B_PER_W)], drop_v)
        pltpu.sync_copy(train_hbm, train_v)
        if has_null:
            pltpu.sync_copy(null_hbm, null_v)
        trn = train_v[...]
        one16 = jnp.full((_L,), 1, jnp.int32)
        zero16 = jnp.full((_L,), 0, jnp.int32)
        zerof16 = jnp.zeros((_L,), jnp.float32)
        cnt = jnp.int32(0)
        for i in range(_NG):
            sl = pl.ds(i * _L, _L)
            dropped = (trn != 0) & (drop_v[sl] != 0)
            local = idx_v[sl] - cbase
            valid = (~dropped) & (local >= 0) & (local < csize)
            idx_v[sl] = jnp.where(valid, local, 0)
            val_v[sl] = jnp.where(valid, one16, zero16)
            dr_v[sl] = jnp.where(dropped, one16, zero16)
            cnt = cnt + plsc.all_reduce_population_count(valid)[0]

        @pl.loop(0, _B_PER_W)
        def _(j):
            for m in range(_OUT_DIM // _L):
                rows_v[j, pl.ds(m * _L, _L)] = zerof16

        @pl.loop(0, _NG)
        def _(g):
            local = idx_v[pl.ds(g * _L, _L)]
            valid = val_v[pl.ds(g * _L, _L)]
            for k in range(_L):
                @pl.when(valid[k] != 0)
                def _():
                    pltpu.async_copy(
                        chunk_hbm.at[pl.ds(local[k], 1), :],
                        rows_v.at[pl.ds(g * _L + k, 1), :], sem)

        if has_null:
            nulls = [null_v[0, pl.ds(m * _L, _L)]
                     for m in range(_OUT_DIM // _L)]

            @pl.loop(0, _NG)
            def _(g):
                dr = dr_v[pl.ds(g * _L, _L)]
                for k in range(_L):
                    @pl.when(dr[k] != 0)
                    def _():
                        for m in range(_OUT_DIM // _L):
                            rows_v[g * _L + k, pl.ds(m * _L, _L)] = nulls[m]

        @pl.loop(0, _B_PER_W)
        def _(j):
            @pl.when(j < cnt)
            def _():
                pltpu.make_async_copy(
                    chunk_hbm.at[pl.ds(0, 1), :],
                    rows_v.at[pl.ds(0, 1), :], sem).wait()

        pltpu.async_copy(rows_v, out_hbm.at[pl.ds(base, _B_PER_W)],
                         sem2).wait()

    return _embed


def kernel(labels, train, force_drop_ids, table):
    labels32 = labels.astype(jnp.int32)
    drop32 = force_drop_ids.astype(jnp.int32)
    train_vec = jnp.full((_L,), jnp.asarray(train, dtype=jnp.int32))
    nullrow = jnp.take(table, jnp.full((1,), _NUM_CLASSES, jnp.int32),
                       axis=0)
    sizes = [_CROWS] * (_NCHUNK - 1) + [_NUM_CLASSES - (_NCHUNK - 1) * _CROWS]
    out = None
    cbase = 0
    for c in range(_NCHUNK):
        csize = sizes[c]
        has_null = c == _NCHUNK - 1
        chunk = lax.slice(table, (cbase, 0), (cbase + csize, _OUT_DIM))
        part = _make_embed(cbase, csize, has_null)(
            labels32, train_vec, drop32, chunk, nullrow)
        out = part if out is None else out + part
        cbase += csize
    return out


# speedup vs baseline: 1.0742x; 1.0185x over previous
import functools

import jax
import jax.numpy as jnp
from jax import lax
from jax.experimental import pallas as pl
from jax.experimental.pallas import tpu as pltpu
from jax.experimental.pallas import tpu_sc as plsc

_NUM_CLASSES = 1000000
_ROWS = _NUM_CLASSES + 1
_OUT_DIM = 64
_BATCH = 16384
_L = 16
_NC = 2
_NS = 16
_NW = _NC * _NS
_B_PER_W = _BATCH // _NW
_NG = _B_PER_W // _L
_NCHUNK = 2
_CROWS = 500000

_mesh = plsc.VectorSubcoreMesh(core_axis_name="c", subcore_axis_name="s")


def _make_embed(cbase, csize, has_null):
    @functools.partial(
        pl.kernel,
        mesh=_mesh,
        out_type=jax.ShapeDtypeStruct((_BATCH, _OUT_DIM), jnp.float32),
        scratch_types=[
            pltpu.VMEM((_B_PER_W,), jnp.int32),
            pltpu.VMEM((_B_PER_W,), jnp.int32),
            pltpu.VMEM((_B_PER_W,), jnp.int32),
            pltpu.VMEM((_B_PER_W,), jnp.int32),
            pltpu.VMEM((_L,), jnp.int32),
            pltpu.VMEM((1, _OUT_DIM), jnp.float32),
            pltpu.VMEM((_B_PER_W, _OUT_DIM), jnp.float32),
            pltpu.SemaphoreType.DMA,
            pltpu.SemaphoreType.DMA,
        ],
        compiler_params=pltpu.CompilerParams(needs_layout_passes=False),
    )
    def _embed(labels_hbm, train_hbm, drop_hbm, chunk_hbm, null_hbm, out_hbm,
               idx_v, drop_v, val_v, dr_v, train_v, null_v, rows_v,
               sem, sem2):
        wid = lax.axis_index("s") * _NC + lax.axis_index("c")
        base = wid * _B_PER_W
        pltpu.sync_copy(labels_hbm.at[pl.ds(base, _B_PER_W)], idx_v)
        pltpu.sync_copy(drop_hbm.at[pl.ds(base, _B_PER_W)], drop_v)
        pltpu.sync_copy(train_hbm, train_v)
        if has_null:
            pltpu.sync_copy(null_hbm, null_v)
        trn = train_v[...]
        one16 = jnp.full((_L,), 1, jnp.int32)
        zero16 = jnp.full((_L,), 0, jnp.int32)
        zerof16 = jnp.zeros((_L,), jnp.float32)
        cnt = jnp.int32(0)
        for i in range(_NG):
            sl = pl.ds(i * _L, _L)
            dropped = (trn != 0) & (drop_v[sl] != 0)
            local = idx_v[sl] - cbase
            valid = (~dropped) & (local >= 0) & (local < csize)
            idx_v[sl] = jnp.where(valid, local, 0)
            val_v[sl] = jnp.where(valid, one16, zero16)
            dr_v[sl] = jnp.where(dropped, one16, zero16)
            cnt = cnt + plsc.all_reduce_population_count(valid)[0]

        @pl.loop(0, _B_PER_W)
        def _(j):
            for m in range(_OUT_DIM // _L):
                rows_v[j, pl.ds(m * _L, _L)] = zerof16

        @pl.loop(0, _NG)
        def _(g):
            local = idx_v[pl.ds(g * _L, _L)]
            valid = val_v[pl.ds(g * _L, _L)]
            for k in range(_L):
                @pl.when(valid[k] != 0)
                def _():
                    pltpu.async_copy(
                        chunk_hbm.at[pl.ds(local[k], 1), :],
                        rows_v.at[pl.ds(g * _L + k, 1), :], sem)

        if has_null:
            nulls = [null_v[0, pl.ds(m * _L, _L)]
                     for m in range(_OUT_DIM // _L)]

            @pl.loop(0, _NG)
            def _(g):
                dr = dr_v[pl.ds(g * _L, _L)]
                for k in range(_L):
                    @pl.when(dr[k] != 0)
                    def _():
                        for m in range(_OUT_DIM // _L):
                            rows_v[g * _L + k, pl.ds(m * _L, _L)] = nulls[m]

        @pl.loop(0, _B_PER_W)
        def _(j):
            @pl.when(j < cnt)
            def _():
                pltpu.make_async_copy(
                    chunk_hbm.at[pl.ds(0, 1), :],
                    rows_v.at[pl.ds(0, 1), :], sem).wait()

        pltpu.async_copy(rows_v, out_hbm.at[pl.ds(base, _B_PER_W)],
                         sem2).wait()

    return _embed


def kernel(labels, train, force_drop_ids, table):
    labels32 = labels.astype(jnp.int32)
    drop32 = force_drop_ids.astype(jnp.int32)
    train_vec = jnp.full((_L,), jnp.asarray(train, dtype=jnp.int32))
    nullrow = jnp.take(table, jnp.full((1,), _NUM_CLASSES, jnp.int32),
                       axis=0)
    sizes = [_CROWS] * (_NCHUNK - 1) + [_NUM_CLASSES - (_NCHUNK - 1) * _CROWS]
    out = None
    cbase = 0
    for c in range(_NCHUNK):
        csize = sizes[c]
        has_null = c == _NCHUNK - 1
        chunk = lax.slice(table, (cbase, 0), (cbase + csize, _OUT_DIM))
        part = _make_embed(cbase, csize, has_null)(
            labels32, train_vec, drop32, chunk, nullrow)
        out = part if out is None else out + part
        cbase += csize
    return out


# speedup vs baseline: 1.5598x; 1.4521x over previous
import functools

import jax
import jax.numpy as jnp
from jax import lax
from jax.experimental import pallas as pl
from jax.experimental.pallas import tpu as pltpu
from jax.experimental.pallas import tpu_sc as plsc

_NUM_CLASSES = 1000000
_ROWS = _NUM_CLASSES + 1
_OUT_DIM = 64
_BATCH = 16384
_L = 16
_NC = 2
_NS = 16
_NW = _NC * _NS
_B_PER_W = _BATCH // _NW
_NG = _B_PER_W // _L
_NCHUNK = 1
_CROWS = 1000000

_mesh = plsc.VectorSubcoreMesh(core_axis_name="c", subcore_axis_name="s")


def _make_embed(cbase, csize, has_null):
    @functools.partial(
        pl.kernel,
        mesh=_mesh,
        out_type=jax.ShapeDtypeStruct((_BATCH, _OUT_DIM), jnp.float32),
        scratch_types=[
            pltpu.VMEM((_B_PER_W,), jnp.int32),
            pltpu.VMEM((_B_PER_W,), jnp.int32),
            pltpu.VMEM((_B_PER_W,), jnp.int32),
            pltpu.VMEM((_B_PER_W,), jnp.int32),
            pltpu.VMEM((_L,), jnp.int32),
            pltpu.VMEM((1, _OUT_DIM), jnp.float32),
            pltpu.VMEM((_B_PER_W, _OUT_DIM), jnp.float32),
            pltpu.SemaphoreType.DMA,
            pltpu.SemaphoreType.DMA,
        ],
        compiler_params=pltpu.CompilerParams(needs_layout_passes=False),
    )
    def _embed(labels_hbm, train_hbm, drop_hbm, chunk_hbm, null_hbm, out_hbm,
               idx_v, drop_v, val_v, dr_v, train_v, null_v, rows_v,
               sem, sem2):
        wid = lax.axis_index("s") * _NC + lax.axis_index("c")
        base = wid * _B_PER_W
        pltpu.sync_copy(labels_hbm.at[pl.ds(base, _B_PER_W)], idx_v)
        pltpu.sync_copy(drop_hbm.at[pl.ds(base, _B_PER_W)], drop_v)
        pltpu.sync_copy(train_hbm, train_v)
        if has_null:
            pltpu.sync_copy(null_hbm, null_v)
        trn = train_v[...]
        one16 = jnp.full((_L,), 1, jnp.int32)
        zero16 = jnp.full((_L,), 0, jnp.int32)
        zerof16 = jnp.zeros((_L,), jnp.float32)
        cnt = jnp.int32(0)
        for i in range(_NG):
            sl = pl.ds(i * _L, _L)
            dropped = (trn != 0) & (drop_v[sl] != 0)
            local = idx_v[sl] - cbase
            valid = (~dropped) & (local >= 0) & (local < csize)
            idx_v[sl] = jnp.where(valid, local, 0)
            val_v[sl] = jnp.where(valid, one16, zero16)
            dr_v[sl] = jnp.where(dropped, one16, zero16)
            cnt = cnt + plsc.all_reduce_population_count(valid)[0]

        @pl.loop(0, _B_PER_W)
        def _(j):
            for m in range(_OUT_DIM // _L):
                rows_v[j, pl.ds(m * _L, _L)] = zerof16

        @pl.loop(0, _NG)
        def _(g):
            local = idx_v[pl.ds(g * _L, _L)]
            valid = val_v[pl.ds(g * _L, _L)]
            for k in range(_L):
                @pl.when(valid[k] != 0)
                def _():
                    pltpu.async_copy(
                        chunk_hbm.at[pl.ds(local[k], 1), :],
                        rows_v.at[pl.ds(g * _L + k, 1), :], sem)

        if has_null:
            nulls = [null_v[0, pl.ds(m * _L, _L)]
                     for m in range(_OUT_DIM // _L)]

            @pl.loop(0, _NG)
            def _(g):
                dr = dr_v[pl.ds(g * _L, _L)]
                for k in range(_L):
                    @pl.when(dr[k] != 0)
                    def _():
                        for m in range(_OUT_DIM // _L):
                            rows_v[g * _L + k, pl.ds(m * _L, _L)] = nulls[m]

        @pl.loop(0, _B_PER_W)
        def _(j):
            @pl.when(j < cnt)
            def _():
                pltpu.make_async_copy(
                    chunk_hbm.at[pl.ds(0, 1), :],
                    rows_v.at[pl.ds(0, 1), :], sem).wait()

        pltpu.async_copy(rows_v, out_hbm.at[pl.ds(base, _B_PER_W)],
                         sem2).wait()

    return _embed


def kernel(labels, train, force_drop_ids, table):
    labels32 = labels.astype(jnp.int32)
    drop32 = force_drop_ids.astype(jnp.int32)
    train_vec = jnp.full((_L,), jnp.asarray(train, dtype=jnp.int32))
    nullrow = jnp.take(table, jnp.full((1,), _NUM_CLASSES, jnp.int32),
                       axis=0)
    sizes = [_CROWS] * (_NCHUNK - 1) + [_NUM_CLASSES - (_NCHUNK - 1) * _CROWS]
    out = None
    cbase = 0
    for c in range(_NCHUNK):
        csize = sizes[c]
        has_null = c == _NCHUNK - 1
        chunk = lax.slice(table, (cbase, 0), (cbase + csize, _OUT_DIM))
        part = _make_embed(cbase, csize, has_null)(
            labels32, train_vec, drop32, chunk, nullrow)
        out = part if out is None else out + part
        cbase += csize
    return out
